# Initial kernel scaffold; baseline (speedup 1.0000x reference)
import jax
import jax.numpy as jnp
from jax.experimental import pallas as pl


def kernel(protein, bn_w, bn_b, Aw, Ab, LNw, LNb, H1w, H1b, H2w, H2b, WFw, WFb, WOw, WOb, WQw, WQb):
    raise NotImplementedError("write your pallas kernel here")


# speedup vs baseline: 4.5938x; 4.5938x over previous
import jax
import jax.numpy as jnp
from jax.experimental import pallas as pl
from jax.experimental.pallas import tpu as pltpu

_BN_SCALE = 1.0 / (1.0 + 1e-5) ** 0.5
_NEG_INF = float("-inf")


def _normalize_rows(x):
    n = jnp.sqrt(jnp.sum(x * x, axis=-1, keepdims=True))
    return x / jnp.maximum(n, 1e-12)


def _router_body(prot_ref, bnw_ref, bnb_ref, Aw_ref, Ab_ref, LNw_ref, LNb_ref,
                 WFw_ref, WFb_ref, WQw_ref, WQb_ref,
                 xa_ref, xb_ref, fa_ref, fb_ref, wsel_ref, mask_ref,
                 feats_scr, keys_scr, qacc_scr):
    M = feats_scr.shape[0]
    BT, H = qacc_scr.shape
    i = pl.program_id(1)

    seg = prot_ref[...] * (bnw_ref[...] * _BN_SCALE) + bnb_ref[...]
    h = jnp.dot(seg, Aw_ref[0], preferred_element_type=jnp.float32) + Ab_ref[0]
    mu = jnp.mean(h, axis=-1, keepdims=True)
    var = jnp.mean((h - mu) ** 2, axis=-1, keepdims=True)
    h = (h - mu) / jnp.sqrt(var + 1e-5) * LNw_ref[0] + LNb_ref[0]
    feat = jnp.maximum(h, 0.0)
    feats_scr[i] = feat
    ku = jnp.dot(feat, WFw_ref[...], preferred_element_type=jnp.float32) + WFb_ref[...]
    keys_scr[i] = _normalize_rows(ku)
    qpart = jnp.dot(feat, WQw_ref[0], preferred_element_type=jnp.float32)

    @pl.when(i == 0)
    def _():
        qacc_scr[...] = qpart + WQb_ref[...]

    @pl.when(i > 0)
    def _():
        qacc_scr[...] = qacc_scr[...] + qpart

    @pl.when(i == M - 1)
    def _():
        q = _normalize_rows(qacc_scr[...])
        sims = [jnp.sum(keys_scr[k] * q, axis=-1, keepdims=True) for k in range(M)]
        sim = jnp.concatenate(sims, axis=-1)
        e = jnp.exp(sim - jnp.max(sim, axis=-1, keepdims=True))
        beta = e / jnp.sum(e, axis=-1, keepdims=True)

        iota = jax.lax.broadcasted_iota(jnp.int32, (BT, M), 1)
        m1 = jnp.max(beta, axis=-1, keepdims=True)
        i1 = jnp.min(jnp.where(beta == m1, iota, M), axis=-1, keepdims=True)
        masked = jnp.where(iota == i1, _NEG_INF, beta)
        m2 = jnp.max(masked, axis=-1, keepdims=True)
        i2 = jnp.min(jnp.where(masked == m2, iota, M), axis=-1, keepdims=True)

        mask_ref[...] = ((iota == i1) | (iota == i2)).astype(jnp.float32)
        denom = m1 + m2 + 1e-8
        wsel_ref[...] = jnp.concatenate([m1 / denom, m2 / denom], axis=-1)

        xa = jnp.zeros((BT, H), jnp.float32)
        xb = jnp.zeros((BT, H), jnp.float32)
        fa = jnp.zeros((BT, H), jnp.float32)
        fb = jnp.zeros((BT, H), jnp.float32)
        for k in range(M):
            sa = (i1 == k).astype(jnp.float32)
            sb = (i2 == k).astype(jnp.float32)
            xa = xa + sa * feats_scr[k]
            xb = xb + sb * feats_scr[k]
            fa = fa + sa * keys_scr[k]
            fb = fb + sb * keys_scr[k]
        xa_ref[...] = xa
        xb_ref[...] = xb
        fa_ref[...] = fa
        fb_ref[...] = fb


def _expert_body(xa_ref, xb_ref, fa_ref, fb_ref, wsel_ref,
                 H1w_ref, H1b_ref, H2w_ref, H2b_ref, WOw_ref, WOb_ref,
                 y_ref, y_scr, sim_scr):
    M = sim_scr.shape[0] // 2
    BT = xa_ref.shape[0]
    O = y_scr.shape[2]
    j = pl.program_id(1)

    for s in range(2):
        xs = xa_ref[...] if s == 0 else xb_ref[...]
        fs = fa_ref[...] if s == 0 else fb_ref[...]
        hj = jnp.dot(xs, H1w_ref[0], preferred_element_type=jnp.float32) + H1b_ref[0]
        hj = jnp.maximum(hj, 0.0)
        y_scr[j + s * M] = jnp.dot(hj, H2w_ref[0], preferred_element_type=jnp.float32) + H2b_ref[0]
        oj = jnp.dot(hj, WOw_ref[...], preferred_element_type=jnp.float32) + WOb_ref[...]
        sim_scr[j + s * M] = jnp.sum(_normalize_rows(oj) * fs, axis=-1, keepdims=True)

    @pl.when(j == M - 1)
    def _():
        acc = jnp.zeros((BT, O), jnp.float32)
        for s in range(2):
            sim = jnp.concatenate([sim_scr[k + s * M] for k in range(M)], axis=-1)
            sim = sim * 2.0
            a = jnp.exp(sim - jnp.max(sim, axis=-1, keepdims=True))
            alpha = a / jnp.sum(a, axis=-1, keepdims=True)
            ysel = jnp.zeros((BT, O), jnp.float32)
            for k in range(M):
                ysel = ysel + alpha[:, k:k + 1] * y_scr[k + s * M]
            acc = acc + wsel_ref[:, s:s + 1] * ysel
        y_ref[...] = acc


def kernel(protein, bn_w, bn_b, Aw, Ab, LNw, LNb, H1w, H1b, H2w, H2b,
           WFw, WFb, WOw, WOb, WQw, WQb):
    B, D = protein.shape
    M, dI, H = Aw.shape
    O = H2w.shape[2]
    BT = 256 if B % 256 == 0 else B
    G = B // BT
    WQw3 = WQw.reshape(M, H, H)
    Ab3 = Ab.reshape(M, 1, H)
    LNw3 = LNw.reshape(M, 1, H)
    LNb3 = LNb.reshape(M, 1, H)
    H1b3 = H1b.reshape(M, 1, H)
    H2b3 = H2b.reshape(M, 1, O)
    f32 = jnp.float32

    xa, xb, fa, fb, wsel, mask = pl.pallas_call(
        _router_body,
        grid=(G, M),
        in_specs=[
            pl.BlockSpec((BT, dI), lambda g, i: (g, i)),
            pl.BlockSpec((dI,), lambda g, i: (i,)),
            pl.BlockSpec((dI,), lambda g, i: (i,)),
            pl.BlockSpec((1, dI, H), lambda g, i: (i, 0, 0)),
            pl.BlockSpec((1, 1, H), lambda g, i: (i, 0, 0)),
            pl.BlockSpec((1, 1, H), lambda g, i: (i, 0, 0)),
            pl.BlockSpec((1, 1, H), lambda g, i: (i, 0, 0)),
            pl.BlockSpec((H, H), lambda g, i: (0, 0)),
            pl.BlockSpec((H,), lambda g, i: (0,)),
            pl.BlockSpec((1, H, H), lambda g, i: (i, 0, 0)),
            pl.BlockSpec((H,), lambda g, i: (0,)),
        ],
        out_specs=[
            pl.BlockSpec((BT, H), lambda g, i: (g, 0)),
            pl.BlockSpec((BT, H), lambda g, i: (g, 0)),
            pl.BlockSpec((BT, H), lambda g, i: (g, 0)),
            pl.BlockSpec((BT, H), lambda g, i: (g, 0)),
            pl.BlockSpec((BT, 2), lambda g, i: (g, 0)),
            pl.BlockSpec((BT, M), lambda g, i: (g, 0)),
        ],
        out_shape=[
            jax.ShapeDtypeStruct((B, H), f32), jax.ShapeDtypeStruct((B, H), f32),
            jax.ShapeDtypeStruct((B, H), f32), jax.ShapeDtypeStruct((B, H), f32),
            jax.ShapeDtypeStruct((B, 2), f32), jax.ShapeDtypeStruct((B, M), f32),
        ],
        scratch_shapes=[
            pltpu.VMEM((M, BT, H), f32),
            pltpu.VMEM((M, BT, H), f32),
            pltpu.VMEM((BT, H), f32),
        ],
    )(protein, bn_w, bn_b, Aw, Ab3, LNw3, LNb3, WFw, WFb, WQw3, WQb)

    y_hat = pl.pallas_call(
        _expert_body,
        grid=(G, M),
        in_specs=[
            pl.BlockSpec((BT, H), lambda g, j: (g, 0)),
            pl.BlockSpec((BT, H), lambda g, j: (g, 0)),
            pl.BlockSpec((BT, H), lambda g, j: (g, 0)),
            pl.BlockSpec((BT, H), lambda g, j: (g, 0)),
            pl.BlockSpec((BT, 2), lambda g, j: (g, 0)),
            pl.BlockSpec((1, H, H), lambda g, j: (j, 0, 0)),
            pl.BlockSpec((1, 1, H), lambda g, j: (j, 0, 0)),
            pl.BlockSpec((1, H, O), lambda g, j: (j, 0, 0)),
            pl.BlockSpec((1, 1, O), lambda g, j: (j, 0, 0)),
            pl.BlockSpec((H, H), lambda g, j: (0, 0)),
            pl.BlockSpec((H,), lambda g, j: (0,)),
        ],
        out_specs=[pl.BlockSpec((BT, O), lambda g, j: (g, 0))],
        out_shape=[jax.ShapeDtypeStruct((B, O), f32)],
        scratch_shapes=[
            pltpu.VMEM((2 * M, BT, O), f32),
            pltpu.VMEM((2 * M, BT, 1), f32),
        ],
    )(xa, xb, fa, fb, wsel, H1w, H1b3, H2w, H2b3, WOw, WOb)[0]

    return (y_hat, mask)
